# baseline (speedup 1.0000x reference)
import functools

import jax
import jax.numpy as jnp
from jax import lax
from jax.experimental import pallas as pl
from jax.experimental.pallas import tpu as pltpu
from jax.experimental.pallas import tpu_sc as plsc

_F32 = jnp.float32
_CHUNK = 128
_DC = 8


def _tc_layer1(x, wl_t, wr_t, b1, bn):
    n, d_in = x.shape
    d_hid = wl_t.shape[1]

    def body(x_ref, wl_ref, wr_ref, b_ref, y1_ref, r1_ref):
        xb = x_ref[...]
        y1_ref[...] = jnp.dot(xb, wl_ref[...], preferred_element_type=_F32)
        r1_ref[...] = jnp.dot(xb, wr_ref[...], preferred_element_type=_F32) + b_ref[...]

    return pl.pallas_call(
        body,
        grid=(n // bn,),
        in_specs=[
            pl.BlockSpec((bn, d_in), lambda i: (i, 0)),
            pl.BlockSpec((d_in, d_hid), lambda i: (0, 0)),
            pl.BlockSpec((d_in, d_hid), lambda i: (0, 0)),
            pl.BlockSpec((1, d_hid), lambda i: (0, 0)),
        ],
        out_specs=[
            pl.BlockSpec((bn, d_hid), lambda i: (i, 0)),
            pl.BlockSpec((bn, d_hid), lambda i: (i, 0)),
        ],
        out_shape=[
            jax.ShapeDtypeStruct((n, d_hid), _F32),
            jax.ShapeDtypeStruct((n, d_hid), _F32),
        ],
    )(x, wl_t, wr_t, b1)


def _tc_layer2(zp1, cp1, r1, wl_t, wr_t, b2, bn):
    n, d_hid = r1.shape
    d_out = wl_t.shape[1]

    def body(zp_ref, cp_ref, r1_ref, wl_ref, wr_ref, b_ref,
             y2_ref, r2_ref, inv_ref):
        zs = zp_ref[0] + zp_ref[1]
        cs = cp_ref[0] + cp_ref[1]
        cnt = jnp.max(cs, axis=1, keepdims=True)
        inv = 1.0 / jnp.maximum(cnt, 1.0)
        h = jnp.maximum(zs * inv + r1_ref[...], 0.0)
        y2_ref[...] = jnp.dot(h, wl_ref[...], preferred_element_type=_F32)
        r2_ref[...] = jnp.dot(h, wr_ref[...], preferred_element_type=_F32) + b_ref[...]
        inv_ref[...] = jnp.broadcast_to(inv, (inv.shape[0], d_out))

    return pl.pallas_call(
        body,
        grid=(n // bn,),
        in_specs=[
            pl.BlockSpec((2, bn, d_hid), lambda i: (0, i, 0)),
            pl.BlockSpec((2, bn, _DC), lambda i: (0, i, 0)),
            pl.BlockSpec((bn, d_hid), lambda i: (i, 0)),
            pl.BlockSpec((d_hid, d_out), lambda i: (0, 0)),
            pl.BlockSpec((d_hid, d_out), lambda i: (0, 0)),
            pl.BlockSpec((1, d_out), lambda i: (0, 0)),
        ],
        out_specs=[
            pl.BlockSpec((bn, d_out), lambda i: (i, 0)),
            pl.BlockSpec((bn, d_out), lambda i: (i, 0)),
            pl.BlockSpec((bn, d_out), lambda i: (i, 0)),
        ],
        out_shape=[
            jax.ShapeDtypeStruct((n, d_out), _F32),
            jax.ShapeDtypeStruct((n, d_out), _F32),
            jax.ShapeDtypeStruct((n, d_out), _F32),
        ],
    )(zp1, cp1, r1, wl_t, wr_t, b2)


def _tc_final(zp2, inv, r2, bn):
    n, d_out = r2.shape

    def body(zp_ref, inv_ref, r2_ref, out_ref):
        out_ref[...] = (zp_ref[0] + zp_ref[1]) * inv_ref[...] + r2_ref[...]

    return pl.pallas_call(
        body,
        grid=(n // bn,),
        in_specs=[
            pl.BlockSpec((2, bn, d_out), lambda i: (0, i, 0)),
            pl.BlockSpec((bn, d_out), lambda i: (i, 0)),
            pl.BlockSpec((bn, d_out), lambda i: (i, 0)),
        ],
        out_specs=pl.BlockSpec((bn, d_out), lambda i: (i, 0)),
        out_shape=jax.ShapeDtypeStruct((n, d_out), _F32),
    )(zp2, inv, r2)


def _sc_agg(n_acc, d, k0, k1, nc, ns, with_count):
    rps = n_acc // ns
    kmax = max(k0, k1)
    mesh = plsc.VectorSubcoreMesh(core_axis_name="c", subcore_axis_name="s")
    out_type = [jax.ShapeDtypeStruct((nc, n_acc, d), _F32)]
    scratch = [
        pltpu.VMEM((kmax, _CHUNK), jnp.int32),
        pltpu.VMEM((kmax, _CHUNK), jnp.int32),
        pltpu.VMEM((2, _CHUNK, d), _F32),
        pltpu.VMEM((rps, d), _F32),
        pltpu.VMEM_SHARED((n_acc, d), _F32),
        pltpu.SemaphoreType.DMA,
        pltpu.SemaphoreType.DMA,
    ]
    if with_count:
        out_type.append(jax.ShapeDtypeStruct((nc, n_acc, _DC), _F32))
        scratch += [
            pltpu.VMEM((_CHUNK, _DC), _F32),
            pltpu.VMEM((rps, _DC), _F32),
            pltpu.VMEM_SHARED((n_acc, _DC), _F32),
        ]

    @functools.partial(
        pl.kernel,
        out_type=out_type,
        mesh=mesh,
        scratch_types=scratch,
        compiler_params=pltpu.CompilerParams(use_tc_tiling_on_sc=False),
    )
    def agg(y_hbm, src_hbm, dst_hbm, z_hbm, *rest):
        if with_count:
            (o_hbm, cz_hbm, out_hbm, cout_hbm, srcv, dstv, rows, zbuf, acc,
             sem0, sem1, ones, czbuf, cacc) = rest
        else:
            out_hbm, srcv, dstv, rows, zbuf, acc, sem0, sem1 = rest
        sems = (sem0, sem1)
        c = lax.axis_index("c")
        s = lax.axis_index("s")
        base = jnp.where(c == 0, s * k0, ns * k0 + s * k1)
        kc = jnp.where(c == 0, k0, k1)

        with jax.named_scope("agg_stage_idx"):
            pltpu.sync_copy(src_hbm.at[pl.ds(base, kmax)], srcv)
            pltpu.sync_copy(dst_hbm.at[pl.ds(base, kmax)], dstv)
            for p in range(2):
                pltpu.async_copy(y_hbm.at[srcv.at[p]], rows.at[p], sems[p])

        with jax.named_scope("agg_zero"):
            pltpu.sync_copy(z_hbm, zbuf)
            pltpu.sync_copy(zbuf, acc.at[pl.ds(s * rps, rps)])
            if with_count:
                pltpu.sync_copy(o_hbm, ones)
                pltpu.sync_copy(cz_hbm, czbuf)
                pltpu.sync_copy(czbuf, cacc.at[pl.ds(s * rps, rps)])
            plsc.subcore_barrier()

        with jax.named_scope("agg_main"):
            def pair(jj, carry):
                for p in range(2):
                    j = 2 * jj + p
                    pltpu.make_async_copy(
                        y_hbm.at[srcv.at[j]], rows.at[p], sems[p]).wait()
                    pltpu.sync_copy(rows.at[p], acc.at[dstv.at[j]], add=True)
                    if with_count:
                        pltpu.sync_copy(ones, cacc.at[dstv.at[j]], add=True)

                    @pl.when(j + 2 < kc)
                    def _refill():
                        pltpu.async_copy(
                            y_hbm.at[srcv.at[j + 2]], rows.at[p], sems[p])
                return carry
            lax.fori_loop(0, kc // 2, pair, 0)
            plsc.subcore_barrier()

        with jax.named_scope("agg_flush"):
            pltpu.sync_copy(acc.at[pl.ds(s * rps, rps)], zbuf)
            pltpu.sync_copy(zbuf, out_hbm.at[c, pl.ds(s * rps, rps)])
            if with_count:
                pltpu.sync_copy(cacc.at[pl.ds(s * rps, rps)], czbuf)
                pltpu.sync_copy(czbuf, cout_hbm.at[c, pl.ds(s * rps, rps)])

    return agg


def kernel(x, edge_index, W1l, b1, W1r, W2l, b2, W2r):
    n, d_in = x.shape
    e = edge_index.shape[1]
    d_hid = W1l.shape[0]
    d_out = W2l.shape[0]

    info = plsc.get_sparse_core_info()
    nc, ns = info.num_cores, info.num_subcores
    nw = nc * ns

    cpw = -(-e // (nw * _CHUNK))
    cpw += cpw % 2
    e_pad = nw * cpw * _CHUNK
    k0_2, k1_2 = 94, 66
    assert k0_2 + k1_2 == 2 * cpw
    slack = max(k0_2, k1_2) - cpw
    n_acc = -(-(n + 1) // 128) * 128
    src = edge_index[0]
    dst = edge_index[1]
    pad = e_pad - e + slack * _CHUNK
    dummy = n + jnp.arange(pad, dtype=jnp.int32) % (n_acc - n)
    src2d = jnp.concatenate(
        [src, jnp.zeros((pad,), jnp.int32)]).reshape(nw * cpw + slack, _CHUNK)
    dst2d = jnp.concatenate([dst, dummy]).reshape(nw * cpw + slack, _CHUNK)

    rps = n_acc // ns
    zeros1 = jnp.zeros((rps, d_hid), _F32)
    zeros2 = jnp.zeros((rps, d_out), _F32)
    ones8 = jnp.ones((_CHUNK, _DC), _F32)
    czeros = jnp.zeros((rps, _DC), _F32)

    bn = 1000
    y1, r1 = _tc_layer1(x, W1l.T, W1r.T, b1.reshape(1, -1), bn)
    zp1, cp1 = _sc_agg(n_acc, d_hid, cpw, cpw, nc, ns, True)(
        y1, src2d, dst2d, zeros1, ones8, czeros)
    y2, r2, inv = _tc_layer2(zp1, cp1, r1, W2l.T, W2r.T, b2.reshape(1, -1), bn)
    (zp2,) = _sc_agg(n_acc, d_out, k0_2, k1_2, nc, ns, False)(
        y2, src2d, dst2d, zeros2)
    return _tc_final(zp2, inv, r2, bn)


# speedup vs baseline: 1.0822x; 1.0822x over previous
import functools

import jax
import jax.numpy as jnp
from jax import lax
from jax.experimental import pallas as pl
from jax.experimental.pallas import tpu as pltpu
from jax.experimental.pallas import tpu_sc as plsc

_F32 = jnp.float32
_CHUNK = 128
_DC = 16


def _tc_layer1(x, wl_t, wr_t, b1, bn):
    n, d_in = x.shape
    d_hid = wl_t.shape[1]

    def body(x_ref, wl_ref, wr_ref, b_ref, y1_ref, r1_ref):
        xb = x_ref[...]
        y1_ref[...] = jnp.dot(xb, wl_ref[...], preferred_element_type=_F32)
        r1_ref[...] = jnp.dot(xb, wr_ref[...], preferred_element_type=_F32) + b_ref[...]

    return pl.pallas_call(
        body,
        grid=(n // bn,),
        in_specs=[
            pl.BlockSpec((bn, d_in), lambda i: (i, 0)),
            pl.BlockSpec((d_in, d_hid), lambda i: (0, 0)),
            pl.BlockSpec((d_in, d_hid), lambda i: (0, 0)),
            pl.BlockSpec((1, d_hid), lambda i: (0, 0)),
        ],
        out_specs=[
            pl.BlockSpec((bn, d_hid), lambda i: (i, 0)),
            pl.BlockSpec((bn, d_hid), lambda i: (i, 0)),
        ],
        out_shape=[
            jax.ShapeDtypeStruct((n, d_hid), _F32),
            jax.ShapeDtypeStruct((n, d_hid), _F32),
        ],
    )(x, wl_t, wr_t, b1)


def _tc_layer2(zp1, cp1, r1, wl_t, wr_t, b2, bn):
    n, d_hid = r1.shape
    d_out = wl_t.shape[1]

    def body(zp_ref, cp_ref, r1_ref, wl_ref, wr_ref, b_ref,
             y2_ref, r2_ref, inv_ref):
        zs = zp_ref[0] + zp_ref[1]
        cs = cp_ref[0] + cp_ref[1]
        cnt = jnp.max(cs, axis=1, keepdims=True)
        inv = 1.0 / jnp.maximum(cnt, 1.0)
        h = jnp.maximum(zs * inv + r1_ref[...], 0.0)
        y2_ref[...] = jnp.dot(h, wl_ref[...], preferred_element_type=_F32)
        r2_ref[...] = jnp.dot(h, wr_ref[...], preferred_element_type=_F32) + b_ref[...]
        inv_ref[...] = jnp.broadcast_to(inv, (inv.shape[0], d_out))

    return pl.pallas_call(
        body,
        grid=(n // bn,),
        in_specs=[
            pl.BlockSpec((2, bn, d_hid), lambda i: (0, i, 0)),
            pl.BlockSpec((2, bn, _DC), lambda i: (0, i, 0)),
            pl.BlockSpec((bn, d_hid), lambda i: (i, 0)),
            pl.BlockSpec((d_hid, d_out), lambda i: (0, 0)),
            pl.BlockSpec((d_hid, d_out), lambda i: (0, 0)),
            pl.BlockSpec((1, d_out), lambda i: (0, 0)),
        ],
        out_specs=[
            pl.BlockSpec((bn, d_out), lambda i: (i, 0)),
            pl.BlockSpec((bn, d_out), lambda i: (i, 0)),
            pl.BlockSpec((bn, d_out), lambda i: (i, 0)),
        ],
        out_shape=[
            jax.ShapeDtypeStruct((n, d_out), _F32),
            jax.ShapeDtypeStruct((n, d_out), _F32),
            jax.ShapeDtypeStruct((n, d_out), _F32),
        ],
    )(zp1, cp1, r1, wl_t, wr_t, b2)


def _tc_final(zp2, inv, r2, bn):
    n, d_out = r2.shape

    def body(zp_ref, inv_ref, r2_ref, out_ref):
        out_ref[...] = (zp_ref[0] + zp_ref[1]) * inv_ref[...] + r2_ref[...]

    return pl.pallas_call(
        body,
        grid=(n // bn,),
        in_specs=[
            pl.BlockSpec((2, bn, d_out), lambda i: (0, i, 0)),
            pl.BlockSpec((bn, d_out), lambda i: (i, 0)),
            pl.BlockSpec((bn, d_out), lambda i: (i, 0)),
        ],
        out_specs=pl.BlockSpec((bn, d_out), lambda i: (i, 0)),
        out_shape=jax.ShapeDtypeStruct((n, d_out), _F32),
    )(zp2, inv, r2)


def _sc_agg(n_acc, d, k0, k1, nc, ns, with_count):
    rps = n_acc // ns
    kmax = max(k0, k1)
    mesh = plsc.VectorSubcoreMesh(core_axis_name="c", subcore_axis_name="s")
    out_type = [jax.ShapeDtypeStruct((nc, n_acc, d), _F32)]
    scratch = [
        pltpu.VMEM((kmax, _CHUNK), jnp.int32),
        pltpu.VMEM((kmax, _CHUNK), jnp.int32),
        pltpu.VMEM((2, _CHUNK, d), _F32),
        pltpu.VMEM((rps, d), _F32),
        pltpu.VMEM_SHARED((n_acc, d), _F32),
        pltpu.SemaphoreType.DMA,
        pltpu.SemaphoreType.DMA,
    ]
    if with_count:
        out_type.append(jax.ShapeDtypeStruct((nc, n_acc, _DC), _F32))
        scratch += [
            pltpu.VMEM((_CHUNK, _DC), _F32),
            pltpu.VMEM((rps, _DC), _F32),
            pltpu.VMEM_SHARED((n_acc, _DC), _F32),
        ]

    @functools.partial(
        pl.kernel,
        out_type=out_type,
        mesh=mesh,
        scratch_types=scratch,
        compiler_params=pltpu.CompilerParams(use_tc_tiling_on_sc=False),
    )
    def agg(y_hbm, src_hbm, dst_hbm, *rest):
        if with_count:
            (out_hbm, cout_hbm, srcv, dstv, rows, zbuf, acc,
             sem0, sem1, ones, czbuf, cacc) = rest
        else:
            out_hbm, srcv, dstv, rows, zbuf, acc, sem0, sem1 = rest
        sems = (sem0, sem1)
        c = lax.axis_index("c")
        s = lax.axis_index("s")
        base = jnp.where(c == 0, s * k0, ns * k0 + s * k1)
        kc = jnp.where(c == 0, k0, k1)

        with jax.named_scope("agg_stage_idx"):
            pltpu.sync_copy(src_hbm.at[pl.ds(base, kmax)], srcv)
            pltpu.sync_copy(dst_hbm.at[pl.ds(base, kmax)], dstv)
            for p in range(2):
                pltpu.async_copy(y_hbm.at[srcv.at[p]], rows.at[p], sems[p])

        with jax.named_scope("agg_zero"):
            def zrow(i, carry):
                for k in range(d // 16):
                    zbuf[i, pl.ds(k * 16, 16)] = jnp.zeros((16,), _F32)
                if with_count:
                    czbuf[i, pl.ds(0, _DC)] = jnp.zeros((_DC,), _F32)
                return carry
            lax.fori_loop(0, rps, zrow, 0)
            pltpu.sync_copy(zbuf, acc.at[pl.ds(s * rps, rps)])
            if with_count:
                def orow(i, carry):
                    ones[i, pl.ds(0, _DC)] = jnp.ones((_DC,), _F32)
                    return carry
                lax.fori_loop(0, _CHUNK, orow, 0)
                pltpu.sync_copy(czbuf, cacc.at[pl.ds(s * rps, rps)])
            plsc.subcore_barrier()

        with jax.named_scope("agg_main"):
            def pair(jj, carry):
                for p in range(2):
                    j = 2 * jj + p
                    pltpu.make_async_copy(
                        y_hbm.at[srcv.at[j]], rows.at[p], sems[p]).wait()
                    pltpu.sync_copy(rows.at[p], acc.at[dstv.at[j]], add=True)
                    if with_count:
                        pltpu.sync_copy(ones, cacc.at[dstv.at[j]], add=True)

                    @pl.when(j + 2 < kc)
                    def _refill():
                        pltpu.async_copy(
                            y_hbm.at[srcv.at[j + 2]], rows.at[p], sems[p])
                return carry
            lax.fori_loop(0, kc // 2, pair, 0)
            plsc.subcore_barrier()

        with jax.named_scope("agg_flush"):
            pltpu.sync_copy(acc.at[pl.ds(s * rps, rps)], zbuf)
            pltpu.sync_copy(zbuf, out_hbm.at[c, pl.ds(s * rps, rps)])
            if with_count:
                pltpu.sync_copy(cacc.at[pl.ds(s * rps, rps)], czbuf)
                pltpu.sync_copy(czbuf, cout_hbm.at[c, pl.ds(s * rps, rps)])

    return agg


def kernel(x, edge_index, W1l, b1, W1r, W2l, b2, W2r):
    n, d_in = x.shape
    e = edge_index.shape[1]
    d_hid = W1l.shape[0]
    d_out = W2l.shape[0]

    info = plsc.get_sparse_core_info()
    nc, ns = info.num_cores, info.num_subcores
    nw = nc * ns

    cpw = -(-e // (nw * _CHUNK))
    cpw += cpw % 2
    e_pad = nw * cpw * _CHUNK
    k0_2, k1_2 = 94, 66
    assert k0_2 + k1_2 == 2 * cpw
    slack = max(k0_2, k1_2) - cpw
    n_acc = -(-(n + 1) // 128) * 128
    src = edge_index[0]
    dst = edge_index[1]
    pad = e_pad - e + slack * _CHUNK
    dummy = n + jnp.arange(pad, dtype=jnp.int32) % (n_acc - n)
    src2d = jnp.concatenate(
        [src, jnp.zeros((pad,), jnp.int32)]).reshape(nw * cpw + slack, _CHUNK)
    dst2d = jnp.concatenate([dst, dummy]).reshape(nw * cpw + slack, _CHUNK)

    bn = 1000
    y1, r1 = _tc_layer1(x, W1l.T, W1r.T, b1.reshape(1, -1), bn)
    zp1, cp1 = _sc_agg(n_acc, d_hid, cpw, cpw, nc, ns, True)(y1, src2d, dst2d)
    y2, r2, inv = _tc_layer2(zp1, cp1, r1, W2l.T, W2r.T, b2.reshape(1, -1), bn)
    (zp2,) = _sc_agg(n_acc, d_out, k0_2, k1_2, nc, ns, False)(y2, src2d, dst2d)
    return _tc_final(zp2, inv, r2, bn)
